# Initial kernel scaffold; baseline (speedup 1.0000x reference)
import jax
import jax.numpy as jnp
from jax.experimental import pallas as pl


def kernel(x, We1, be1, We2, be2, Wx1, bx1, Wx2, bx2, Wxo, bxo, Winf, binf, Wh1, bh1, Wh2, bh2, Who, bho, Wv, Ws, bs):
    raise NotImplementedError("write your pallas kernel here")


# speedup vs baseline: 17.1170x; 17.1170x over previous
import functools

import jax
import jax.numpy as jnp
import numpy as np
from jax.experimental import pallas as pl

B = 8
N = 128
V = 20
F = 128
M = 128
NB = 3
RV = 8
RS = 128

R = 16
T = N // R

_INV_N1 = 1.0 / (N - 1)
_INV_SQRT_N1 = float(1.0 / np.sqrt(N - 1.0))


def _silu(x):
    return x * jax.nn.sigmoid(x)


def _dot(a, b):
    return jnp.dot(a, b, preferred_element_type=jnp.float32)


def _block_body(last, vec_ref, h_ref, We1v_ref, We1s_ref, We1r_ref, be1_ref,
                We2_ref, be2_ref, Wx1_ref, bx1_ref, Wx2_ref, bx2_ref,
                Wxo_ref, bxo_ref, Winf_ref, binf_ref, Wh1_ref, bh1_ref,
                Wh2_ref, bh2_ref, Who_ref, bho_ref, *rest):
    if last:
        Wv_ref, Ws_ref, bs_ref, vr_ref, sc_ref = rest
    else:
        vec_out_ref, h_out_ref = rest

    ti = pl.program_id(1)
    i0 = ti * R

    vx = vec_ref[0, 0]
    vy = vec_ref[0, 1]
    vz = vec_ref[0, 2]
    h_all = h_ref[0]
    h_r = h_ref[0, pl.ds(i0, R), :]
    vrx = vec_ref[0, 0, pl.ds(i0, R), :]
    vry = vec_ref[0, 1, pl.ds(i0, R), :]
    vrz = vec_ref[0, 2, pl.ds(i0, R), :]

    dx = vrx[:, None, :] - vx[None, :, :]
    dy = vry[:, None, :] - vy[None, :, :]
    dz = vrz[:, None, :] - vz[None, :, :]
    len2 = dx * dx + dy * dy + dz * dz + 1e-20
    length = jnp.sqrt(len2)

    hs = _dot(h_all, We1s_ref[...])
    hr = _dot(h_r, We1r_ref[...])
    m1 = _dot(len2.reshape(R * N, V), We1v_ref[...]).reshape(R, N, M)
    m1 = _silu(m1 + hs[None, :, :] + hr[:, None, :] + be1_ref[...][None, :, :])
    m_ij = _silu(_dot(m1.reshape(R * N, M), We2_ref[...]) + be2_ref[...])

    p = _silu(_dot(m_ij, Wx1_ref[...]) + bx1_ref[...])
    p = _silu(_dot(p, Wx2_ref[...]) + bx2_ref[...])
    px = _dot(p, Wxo_ref[...]) + bxo_ref[...]
    einf = jax.nn.sigmoid(_dot(m_ij, Winf_ref[...]) + binf_ref[...])

    rows = i0 + jax.lax.broadcasted_iota(jnp.int32, (R, N), 0)
    cols = jax.lax.broadcasted_iota(jnp.int32, (R, N), 1)
    mask = (rows != cols).astype(jnp.float32)

    w = px.reshape(R, N, V) / (1.0 + length) * mask[:, :, None]
    wsum = jnp.sum(w, axis=1)
    t2x = jnp.sum(w * vx[None, :, :], axis=1)
    t2y = jnp.sum(w * vy[None, :, :], axis=1)
    t2z = jnp.sum(w * vz[None, :, :], axis=1)
    nvx = vrx + (vrx * wsum - t2x) * _INV_N1
    nvy = vry + (vry * wsum - t2y) * _INV_N1
    nvz = vrz + (vrz * wsum - t2z) * _INV_N1

    em = einf.reshape(R, N, 1) * mask[:, :, None]
    m_i = jnp.sum(m_ij.reshape(R, N, M) * em, axis=1) * _INV_SQRT_N1

    hcat = jnp.concatenate([m_i, h_r], axis=1)
    q = _silu(_dot(hcat, Wh1_ref[...]) + bh1_ref[...])
    q = _silu(_dot(q, Wh2_ref[...]) + bh2_ref[...])
    h_new = _dot(q, Who_ref[...]) + bho_ref[...] + h_r

    if last:
        z = h_new - jnp.max(h_new, axis=1, keepdims=True)
        ez = jnp.exp(z)
        sm = ez / jnp.sum(ez, axis=1, keepdims=True)
        sc_ref[0] = _dot(sm, Ws_ref[...]) + bs_ref[...]
        Wv = Wv_ref[...]
        vr_ref[0, 0] = _dot(nvx, Wv)
        vr_ref[0, 1] = _dot(nvy, Wv)
        vr_ref[0, 2] = _dot(nvz, Wv)
    else:
        vec_out_ref[0, 0] = nvx
        vec_out_ref[0, 1] = nvy
        vec_out_ref[0, 2] = nvz
        h_out_ref[0] = h_new


def _full(shape):
    nd = len(shape)
    return pl.BlockSpec(shape, lambda bi, ti, _n=nd: (0,) * _n)


_IN_SPECS = [
    pl.BlockSpec((1, 3, N, V), lambda bi, ti: (bi, 0, 0, 0)),
    pl.BlockSpec((1, N, F), lambda bi, ti: (bi, 0, 0)),
    _full((V, M)), _full((F, M)), _full((F, M)), _full((1, M)),
    _full((M, M)), _full((1, M)),
    _full((M, M)), _full((1, M)),
    _full((M, M)), _full((1, M)),
    _full((M, V)), _full((1, V)),
    _full((M, 1)), _full((1, 1)),
    _full((M + F, M)), _full((1, M)),
    _full((M, M)), _full((1, M)),
    _full((M, F)), _full((1, F)),
]

_MID_CALL = pl.pallas_call(
    functools.partial(_block_body, False),
    grid=(B, T),
    in_specs=_IN_SPECS,
    out_specs=[
        pl.BlockSpec((1, 3, R, V), lambda bi, ti: (bi, 0, ti, 0)),
        pl.BlockSpec((1, R, F), lambda bi, ti: (bi, ti, 0)),
    ],
    out_shape=[
        jax.ShapeDtypeStruct((B, 3, N, V), jnp.float32),
        jax.ShapeDtypeStruct((B, N, F), jnp.float32),
    ],
)

_LAST_CALL = pl.pallas_call(
    functools.partial(_block_body, True),
    grid=(B, T),
    in_specs=_IN_SPECS + [_full((V, RV)), _full((F, RS)), _full((1, RS))],
    out_specs=[
        pl.BlockSpec((1, 3, R, RV), lambda bi, ti: (bi, 0, ti, 0)),
        pl.BlockSpec((1, R, RS), lambda bi, ti: (bi, ti, 0)),
    ],
    out_shape=[
        jax.ShapeDtypeStruct((B, 3, N, RV), jnp.float32),
        jax.ShapeDtypeStruct((B, N, RS), jnp.float32),
    ],
)


def kernel(x, We1, be1, We2, be2, Wx1, bx1, Wx2, bx2, Wxo, bxo, Winf, binf,
           Wh1, bh1, Wh2, bh2, Who, bho, Wv, Ws, bs):
    vec0 = x - jnp.mean(x, axis=1, keepdims=True)
    vec = jnp.broadcast_to(
        jnp.transpose(vec0, (0, 2, 1))[:, :, :, None], (B, 3, N, V))
    h = jnp.zeros((B, N, F), jnp.float32)

    for b in range(NB):
        args = (
            vec, h,
            We1[b, :V], We1[b, V:V + F], We1[b, V + F:], be1[b][None],
            We2[b], be2[b][None], Wx1[b], bx1[b][None], Wx2[b], bx2[b][None],
            Wxo[b], bxo[b][None], Winf[b], binf[b][None],
            Wh1[b], bh1[b][None], Wh2[b], bh2[b][None], Who[b], bho[b][None],
        )
        if b < NB - 1:
            vec, h = _MID_CALL(*args)
        else:
            vr, sc = _LAST_CALL(*args, Wv, Ws, bs[None])

    vec_read = jnp.transpose(vr, (0, 2, 3, 1))
    return vec_read, sc


# speedup vs baseline: 21.7384x; 1.2700x over previous
import functools

import jax
import jax.numpy as jnp
import numpy as np
from jax.experimental import pallas as pl

B = 8
N = 128
V = 20
F = 128
M = 128
NB = 3
RV = 8
RS = 128

R = 64
T = N // R

_INV_N1 = 1.0 / (N - 1)
_INV_SQRT_N1 = float(1.0 / np.sqrt(N - 1.0))


def _silu(x):
    return x * jax.nn.sigmoid(x)


def _dot(a, b):
    return jnp.dot(a, b, preferred_element_type=jnp.float32)


def _block_body(last, vec_ref, h_ref, We1_ref, be1_ref,
                We2_ref, be2_ref, Wx1_ref, bx1_ref, Wx2_ref, bx2_ref,
                Wxo_ref, bxo_ref, Winf_ref, binf_ref, Wh1_ref, bh1_ref,
                Wh2_ref, bh2_ref, Who_ref, bho_ref, *rest):
    if last:
        Wv_ref, Ws_ref, bs_ref, vr_ref, sc_ref = rest
    else:
        vec_out_ref, h_out_ref = rest

    ti = pl.program_id(1)
    i0 = ti * R

    vx = vec_ref[0, 0]
    vy = vec_ref[0, 1]
    vz = vec_ref[0, 2]
    h_all = h_ref[0]
    h_r = h_ref[0, pl.ds(i0, R), :]
    vrx = vec_ref[0, 0, pl.ds(i0, R), :]
    vry = vec_ref[0, 1, pl.ds(i0, R), :]
    vrz = vec_ref[0, 2, pl.ds(i0, R), :]

    vx4 = jnp.concatenate([vx, vx, vx, vx], axis=1)
    vy4 = jnp.concatenate([vy, vy, vy, vy], axis=1)
    vz4 = jnp.concatenate([vz, vz, vz, vz], axis=1)
    vrx4 = jnp.concatenate([vrx, vrx, vrx, vrx], axis=1)
    vry4 = jnp.concatenate([vry, vry, vry, vry], axis=1)
    vrz4 = jnp.concatenate([vrz, vrz, vrz, vrz], axis=1)
    dx = vrx4[:, None, :] - vx4[None, :, :]
    dy = vry4[:, None, :] - vy4[None, :, :]
    dz = vrz4[:, None, :] - vz4[None, :, :]
    len2_4 = dx * dx + dy * dy + dz * dz + 1e-20
    length4 = jnp.sqrt(len2_4)
    len2 = len2_4[:, :, :V]

    hs_rep = jnp.broadcast_to(h_all[None, :, :], (R, N, F)).reshape(R * N, F)
    hr_rep = jnp.broadcast_to(h_r[:, None, :], (R, N, F)).reshape(R * N, F)
    ef = jnp.concatenate([len2.reshape(R * N, V), hs_rep, hr_rep], axis=1)
    m1 = _silu(_dot(ef, We1_ref[...]) + be1_ref[...])
    m_ij = _silu(_dot(m1, We2_ref[...]) + be2_ref[...])

    p = _silu(_dot(m_ij, Wx1_ref[...]) + bx1_ref[...])
    p = _silu(_dot(p, Wx2_ref[...]) + bx2_ref[...])
    px = _dot(p, Wxo_ref[...]) + bxo_ref[...]
    einf = jax.nn.sigmoid(_dot(m_ij, Winf_ref[...]) + binf_ref[...])
    m_ij = m_ij.reshape(R, N, M)

    rows = i0 + jax.lax.broadcasted_iota(jnp.int32, (R, N), 0)
    cols = jax.lax.broadcasted_iota(jnp.int32, (R, N), 1)
    mask = (rows != cols).astype(jnp.float32)

    w4 = px.reshape(R, N, 4 * V) / (1.0 + length4) * mask[:, :, None]
    mult = jnp.concatenate(
        [jnp.ones_like(vx), vx, vy, vz], axis=1)
    red = jnp.sum(w4 * mult[None, :, :], axis=1)
    wsum = red[:, 0 * V:1 * V]
    t2x = red[:, 1 * V:2 * V]
    t2y = red[:, 2 * V:3 * V]
    t2z = red[:, 3 * V:4 * V]
    nvx = vrx + (vrx * wsum - t2x) * _INV_N1
    nvy = vry + (vry * wsum - t2y) * _INV_N1
    nvz = vrz + (vrz * wsum - t2z) * _INV_N1

    em = einf.reshape(R, N, 1) * mask[:, :, None]
    m_i = jnp.sum(m_ij * em, axis=1) * _INV_SQRT_N1

    hcat = jnp.concatenate([m_i, h_r], axis=1)
    q = _silu(_dot(hcat, Wh1_ref[...]) + bh1_ref[...])
    q = _silu(_dot(q, Wh2_ref[...]) + bh2_ref[...])
    h_new = _dot(q, Who_ref[...]) + bho_ref[...] + h_r

    if last:
        z = h_new - jnp.max(h_new, axis=1, keepdims=True)
        ez = jnp.exp(z)
        sm = ez / jnp.sum(ez, axis=1, keepdims=True)
        sc_ref[0] = _dot(sm, Ws_ref[...]) + bs_ref[...]
        Wv = Wv_ref[...]
        vr_ref[0, 0] = _dot(nvx, Wv)
        vr_ref[0, 1] = _dot(nvy, Wv)
        vr_ref[0, 2] = _dot(nvz, Wv)
    else:
        vec_out_ref[0, 0] = nvx
        vec_out_ref[0, 1] = nvy
        vec_out_ref[0, 2] = nvz
        h_out_ref[0] = h_new


def _full(shape):
    nd = len(shape)
    return pl.BlockSpec(shape, lambda bi, ti, _n=nd: (0,) * _n)


_IN_SPECS = [
    pl.BlockSpec((1, 3, N, V), lambda bi, ti: (bi, 0, 0, 0)),
    pl.BlockSpec((1, N, F), lambda bi, ti: (bi, 0, 0)),
    _full((V + 2 * F, M)), _full((1, M)),
    _full((M, M)), _full((1, M)),
    _full((M, M)), _full((1, M)),
    _full((M, M)), _full((1, M)),
    _full((M, 4 * V)), _full((1, 4 * V)),
    _full((M, 1)), _full((1, 1)),
    _full((M + F, M)), _full((1, M)),
    _full((M, M)), _full((1, M)),
    _full((M, F)), _full((1, F)),
]

_MID_CALL = pl.pallas_call(
    functools.partial(_block_body, False),
    grid=(B, T),
    in_specs=_IN_SPECS,
    out_specs=[
        pl.BlockSpec((1, 3, R, V), lambda bi, ti: (bi, 0, ti, 0)),
        pl.BlockSpec((1, R, F), lambda bi, ti: (bi, ti, 0)),
    ],
    out_shape=[
        jax.ShapeDtypeStruct((B, 3, N, V), jnp.float32),
        jax.ShapeDtypeStruct((B, N, F), jnp.float32),
    ],
)

_LAST_CALL = pl.pallas_call(
    functools.partial(_block_body, True),
    grid=(B, T),
    in_specs=_IN_SPECS + [_full((V, RV)), _full((F, RS)), _full((1, RS))],
    out_specs=[
        pl.BlockSpec((1, 3, R, RV), lambda bi, ti: (bi, 0, ti, 0)),
        pl.BlockSpec((1, R, RS), lambda bi, ti: (bi, ti, 0)),
    ],
    out_shape=[
        jax.ShapeDtypeStruct((B, 3, N, RV), jnp.float32),
        jax.ShapeDtypeStruct((B, N, RS), jnp.float32),
    ],
)


def kernel(x, We1, be1, We2, be2, Wx1, bx1, Wx2, bx2, Wxo, bxo, Winf, binf,
           Wh1, bh1, Wh2, bh2, Who, bho, Wv, Ws, bs):
    vec0 = x - jnp.mean(x, axis=1, keepdims=True)
    vec = jnp.broadcast_to(
        jnp.transpose(vec0, (0, 2, 1))[:, :, :, None], (B, 3, N, V))
    h = jnp.zeros((B, N, F), jnp.float32)

    for b in range(NB):
        args = (
            vec, h,
            We1[b], be1[b][None],
            We2[b], be2[b][None], Wx1[b], bx1[b][None], Wx2[b], bx2[b][None],
            jnp.tile(Wxo[b], (1, 4)), jnp.tile(bxo[b][None], (1, 4)),
            Winf[b], binf[b][None],
            Wh1[b], bh1[b][None], Wh2[b], bh2[b][None], Who[b], bho[b][None],
        )
        if b < NB - 1:
            vec, h = _MID_CALL(*args)
        else:
            vr, sc = _LAST_CALL(*args, Wv, Ws, bs[None])

    vec_read = jnp.transpose(vr, (0, 2, 3, 1))
    return vec_read, sc
